# baseline (speedup 1.0000x reference)
import functools

import jax
import jax.numpy as jnp
from jax import lax
from jax.experimental import pallas as pl
from jax.experimental.pallas import tpu as pltpu
from jax.experimental.pallas import tpu_sc as plsc

B, L = 4096, 200
VOCAB, D = 1000000, 64
EPS = 1e-5

_NC, _NS = 2, 16
_NW = _NC * _NS
_RPW = B // _NW
_IPW = _RPW * L
_NBUF = 2


def _sc_pool_body(idx_hbm, table_hbm, out_hbm, idx_v, rows_v, pooled_v,
                  sem0, sem1):
    wid = lax.axis_index("s") * _NC + lax.axis_index("c")
    rbase = wid * _RPW

    pltpu.sync_copy(idx_hbm.at[pl.ds(rbase, _RPW)], idx_v)

    sems = [sem0, sem1]

    def start_row(r, buf):
        pltpu.async_copy(
            table_hbm.at[idx_v.at[r, pl.ds(0, 128)]],
            rows_v.at[buf].at[pl.ds(0, 128)],
            sems[buf],
        )
        pltpu.async_copy(
            table_hbm.at[idx_v.at[r, pl.ds(128, 72)]],
            rows_v.at[buf].at[pl.ds(128, 72)],
            sems[buf],
        )

    def wait_row(buf):
        pltpu.make_async_copy(
            table_hbm.at[idx_v.at[0, pl.ds(0, 128)]],
            rows_v.at[buf].at[pl.ds(0, 128)],
            sems[buf],
        ).wait()
        pltpu.make_async_copy(
            table_hbm.at[idx_v.at[0, pl.ds(128, 72)]],
            rows_v.at[buf].at[pl.ds(128, 72)],
            sems[buf],
        ).wait()

    def accum_row(r, buf):
        def acc_body(l, accs):
            return tuple(
                a + rows_v[buf, l, pl.ds(16 * j, 16)]
                for j, a in enumerate(accs)
            )
        zeros = tuple(jnp.zeros((16,), jnp.float32) for _ in range(4))
        accs = lax.fori_loop(0, L, acc_body, zeros, unroll=4)
        for j in range(4):
            pooled_v[r, pl.ds(16 * j, 16)] = accs[j]

    start_row(0, 0)

    def outer(i, _):
        for b in range(_NBUF):
            r = i * _NBUF + b
            wait_row(b)

            @pl.when(r + 1 < _RPW)
            def _():
                start_row(r + 1, (b + 1) % _NBUF)

            accum_row(r, b)
        return 0

    lax.fori_loop(0, _RPW // _NBUF, outer, 0)

    pltpu.sync_copy(pooled_v, out_hbm.at[pl.ds(wid * _RPW, _RPW)])


_sc_pool = functools.partial(
    pl.kernel,
    out_type=jax.ShapeDtypeStruct((B, D), jnp.float32),
    mesh=plsc.VectorSubcoreMesh(core_axis_name="c", subcore_axis_name="s",
                                num_cores=_NC, num_subcores=_NS),
    scratch_types=[
        pltpu.VMEM((_RPW, L), jnp.int32),
        pltpu.VMEM((_NBUF, L, D), jnp.float32),
        pltpu.VMEM((_RPW, D), jnp.float32),
        pltpu.SemaphoreType.DMA,
        pltpu.SemaphoreType.DMA,
    ],
    compiler_params=pltpu.CompilerParams(use_tc_tiling_on_sc=False),
)(_sc_pool_body)


def _tc_body(sum_ref, w_ref, b_ref, gamma_ref, beta_ref, out_ref):
    pooled = sum_ref[...] * (1.0 / L)
    dense = lax.dot_general(pooled, w_ref[...], (((1,), (1,)), ((), ())),
                            preferred_element_type=jnp.float32)
    dense = dense + b_ref[...]
    mean = jnp.mean(dense, axis=0, keepdims=True)
    cent = dense - mean
    var = jnp.mean(cent * cent, axis=0, keepdims=True)
    normalized = cent / jnp.sqrt(var + EPS) * gamma_ref[...] + beta_ref[...]
    norm = jnp.sqrt(jnp.sum(normalized * normalized, axis=1, keepdims=True))
    out_ref[...] = normalized / norm


def kernel(inputs, table, W, b, gamma, beta):
    pooled_sum = _sc_pool(inputs.astype(jnp.int32), table)
    out = pl.pallas_call(
        _tc_body,
        out_shape=jax.ShapeDtypeStruct((B, D), jnp.float32),
    )(pooled_sum, W, b.reshape(1, D), gamma.reshape(1, D),
      beta.reshape(1, D))
    return out


# speedup vs baseline: 1.1143x; 1.1143x over previous
import functools

import jax
import jax.numpy as jnp
from jax import lax
from jax.experimental import pallas as pl
from jax.experimental.pallas import tpu as pltpu
from jax.experimental.pallas import tpu_sc as plsc

B, L = 4096, 200
VOCAB, D = 1000000, 64
EPS = 1e-5

_NC, _NS = 2, 16
_NW = _NC * _NS
_RPW = B // _NW
_IPW = _RPW * L
_NBUF = 4


def _sc_pool_body(idx_hbm, table_hbm, out_hbm, idx_v, rows_v, pooled_v,
                  sem0, sem1, sem2, sem3):
    wid = lax.axis_index("s") * _NC + lax.axis_index("c")
    rbase = wid * _RPW

    pltpu.sync_copy(idx_hbm.at[pl.ds(rbase, _RPW)], idx_v)

    sems = [sem0, sem1, sem2, sem3]

    def start_row(r, buf):
        pltpu.async_copy(
            table_hbm.at[idx_v.at[r, pl.ds(0, 128)]],
            rows_v.at[buf].at[pl.ds(0, 128)],
            sems[buf],
        )
        pltpu.async_copy(
            table_hbm.at[idx_v.at[r, pl.ds(128, 72)]],
            rows_v.at[buf].at[pl.ds(128, 72)],
            sems[buf],
        )

    def wait_row(buf):
        pltpu.make_async_copy(
            table_hbm.at[idx_v.at[0, pl.ds(0, 128)]],
            rows_v.at[buf].at[pl.ds(0, 128)],
            sems[buf],
        ).wait()
        pltpu.make_async_copy(
            table_hbm.at[idx_v.at[0, pl.ds(128, 72)]],
            rows_v.at[buf].at[pl.ds(128, 72)],
            sems[buf],
        ).wait()

    def accum_row(r, buf):
        def acc_body(l, accs):
            return tuple(
                a + rows_v[buf, l, pl.ds(16 * j, 16)]
                for j, a in enumerate(accs)
            )
        zeros = tuple(jnp.zeros((16,), jnp.float32) for _ in range(4))
        accs = lax.fori_loop(0, L, acc_body, zeros, unroll=4)
        for j in range(4):
            pooled_v[r, pl.ds(16 * j, 16)] = accs[j]

    for b in range(_NBUF - 1):
        start_row(b, b)

    def outer(i, _):
        for b in range(_NBUF):
            r = i * _NBUF + b

            @pl.when(r + _NBUF - 1 < _RPW)
            def _():
                start_row(r + _NBUF - 1, (b + _NBUF - 1) % _NBUF)

            wait_row(b)
            accum_row(r, b)
        return 0

    lax.fori_loop(0, _RPW // _NBUF, outer, 0)

    pltpu.sync_copy(pooled_v, out_hbm.at[pl.ds(wid * _RPW, _RPW)])


_sc_pool = functools.partial(
    pl.kernel,
    out_type=jax.ShapeDtypeStruct((B, D), jnp.float32),
    mesh=plsc.VectorSubcoreMesh(core_axis_name="c", subcore_axis_name="s",
                                num_cores=_NC, num_subcores=_NS),
    scratch_types=[
        pltpu.VMEM((_RPW, L), jnp.int32),
        pltpu.VMEM((_NBUF, L, D), jnp.float32),
        pltpu.VMEM((_RPW, D), jnp.float32),
        pltpu.SemaphoreType.DMA,
        pltpu.SemaphoreType.DMA,
        pltpu.SemaphoreType.DMA,
        pltpu.SemaphoreType.DMA,
    ],
    compiler_params=pltpu.CompilerParams(use_tc_tiling_on_sc=False),
)(_sc_pool_body)


def _tc_body(sum_ref, w_ref, b_ref, gamma_ref, beta_ref, out_ref):
    pooled = sum_ref[...] * (1.0 / L)
    dense = lax.dot_general(pooled, w_ref[...], (((1,), (1,)), ((), ())),
                            preferred_element_type=jnp.float32)
    dense = dense + b_ref[...]
    mean = jnp.mean(dense, axis=0, keepdims=True)
    cent = dense - mean
    var = jnp.mean(cent * cent, axis=0, keepdims=True)
    normalized = cent / jnp.sqrt(var + EPS) * gamma_ref[...] + beta_ref[...]
    norm = jnp.sqrt(jnp.sum(normalized * normalized, axis=1, keepdims=True))
    out_ref[...] = normalized / norm


def kernel(inputs, table, W, b, gamma, beta):
    pooled_sum = _sc_pool(inputs.astype(jnp.int32), table)
    out = pl.pallas_call(
        _tc_body,
        out_shape=jax.ShapeDtypeStruct((B, D), jnp.float32),
    )(pooled_sum, W, b.reshape(1, D), gamma.reshape(1, D),
      beta.reshape(1, D))
    return out
